# baseline (speedup 1.0000x reference)
import functools

import jax
import jax.numpy as jnp
from jax import lax
from jax.experimental import pallas as pl
from jax.experimental.pallas import tpu as pltpu
from jax.experimental.pallas import tpu_sc as plsc

K = 32
TEMP = 0.1
STEPS = 3
LN_EPS = 1e-5

NC = 2
NS = 16
NW = NC * NS
CAP = 34


def _silu(v):
    return v * (1.0 / (1.0 + jnp.exp(-v)))


def _knn_body(x_ref, w_in_ref, b_in_ref, w1x_ref, b1_ref, p_ref, m_ref,
              pb_ref, h0_ref, xt_ref, *, rb, n, k):
    xb = x_ref[pl.ds(pl.program_id(0) * rb, rb), :]
    xall = x_ref[...]
    nb = jnp.sum(xb * xb, axis=1, keepdims=True)
    na = jnp.sum(xall * xall, axis=1)[None, :]
    cross = lax.dot_general(
        xb, xall, (((1,), (1,)), ((), ())), preferred_element_type=jnp.float32
    )
    sq = jnp.maximum(nb + na - 2.0 * cross, 0.0)
    dist = jnp.sqrt(jnp.maximum(sq, 1e-12))
    rows = pl.program_id(0) * rb + lax.broadcasted_iota(jnp.int32, (rb, n), 0)
    cols = lax.broadcasted_iota(jnp.int32, (rb, n), 1)
    dist = jnp.where(rows == cols, jnp.inf, dist)

    bits = lax.bitcast_convert_type(dist, jnp.int32)

    def search_body(_, lohi):
        lo, hi = lohi
        mid = lo + lax.div(hi - lo, 2)
        cnt = jnp.sum((bits <= mid[:, None]).astype(jnp.float32), axis=1)
        ge = cnt >= float(k)
        return jnp.where(ge, lo, mid + 1), jnp.where(ge, mid, hi)

    lo0 = jnp.zeros((rb,), jnp.int32)
    hi0 = jnp.full((rb,), jnp.int32(0x7F800000))
    _, hi = lax.fori_loop(0, 31, search_body, (lo0, hi0))
    t = lax.bitcast_convert_type(hi, jnp.float32)

    m = jnp.min(dist, axis=1, keepdims=True)
    mask = dist <= t[:, None]
    e = jnp.where(mask, jnp.exp(-(dist - m) / TEMP), 0.0)
    den = jnp.sum(e, axis=1, keepdims=True)
    m_ref[...] = e / den

    def mm(a, b):
        return lax.dot_general(
            a, b, (((1,), (0,)), ((), ())), preferred_element_type=jnp.float32
        )

    pb_ref[...] = mm(mask.astype(jnp.float32), p_ref[...])

    h0_ref[...] = mm(xb, w_in_ref[...]) + b_in_ref[...]
    xt_ref[...] = mm(xb, w1x_ref[...]) + b1_ref[...]


def _step_body(agg_ref, hb_ref, xt_ref, w1h_ref, w1a_ref, w2_ref, b2_ref,
               g_ref, b_ref, out_ref):
    def mm(a, b):
        return lax.dot_general(
            a, b, (((1,), (0,)), ((), ())), preferred_element_type=jnp.float32
        )

    hb = hb_ref[...]
    z = mm(hb, w1h_ref[...]) + mm(agg_ref[...], w1a_ref[...]) + xt_ref[...]
    msg = mm(_silu(z), w2_ref[...]) + b2_ref[...]
    pre = hb + msg
    mu = jnp.mean(pre, axis=1, keepdims=True)
    var = jnp.mean((pre - mu) ** 2, axis=1, keepdims=True)
    out_ref[...] = (pre - mu) / jnp.sqrt(var + LN_EPS) * g_ref[...] + b_ref[...]


def _readout_body(hb_ref, xb_ref, w1h_ref, w1x_ref, b1_ref, w2_ref, b2_ref,
                  out_ref):
    def mm(a, b):
        return lax.dot_general(
            a, b, (((1,), (0,)), ((), ())), preferred_element_type=jnp.float32
        )

    z = mm(hb_ref[...], w1h_ref[...]) + mm(xb_ref[...], w1x_ref[...]) + b1_ref[...]
    h1 = _silu(z)
    ro = jnp.sum(h1 * w2_ref[...].T, axis=1) + b2_ref[0]
    out_ref[...] = jnp.maximum(ro, 0.0) + jnp.log1p(jnp.exp(-jnp.abs(ro)))


def _compact_body(mf_hbm, pb_hbm, idx_hbm, w_hbm, pb0, pb1,
                  wvb, wib, li, fi2d, oi, ow, psem0, psem1, gsem,
                  *, n, rpw):
    wid = lax.axis_index("s") * NC + lax.axis_index("c")
    base = wid * rpw
    lane = lax.iota(jnp.int32, 16)
    psems = (psem0, psem1)
    pbufs = (pb0, pb1)
    nw16 = n // 16

    pltpu.async_copy(pb_hbm.at[base], pb0, psem0)

    def do_row(r, b):
        @pl.when(r + 1 < rpw)
        def _start_next():
            pltpu.async_copy(pb_hbm.at[base + r + 1], pbufs[1 - b],
                             psems[1 - b])

        pltpu.make_async_copy(pb_hbm.at[base + r], pbufs[b], psems[b]).wait()
        pbuf = pbufs[b]

        def word_body(c4, offw):
            for u in range(4):
                c = c4 * 4 + u
                wq = pbuf[pl.ds(c * 16, 16)].astype(jnp.int32)
                nz = (wq != 0) & (offw < CAP)
                tgt = offw * 16 + lane
                plsc.store_scatter(wib, [tgt], c * 16 + lane, mask=nz)
                plsc.store_scatter(wvb, [tgt], wq, mask=nz)
                offw = offw + nz.astype(jnp.int32)
            return offw

        offw = lax.fori_loop(0, nw16 // 64, word_body,
                             jnp.zeros((16,), jnp.int32))
        nmax = lax.reduce_max(offw, (0,))

        def slot_body(s, off):
            wvec = wvb[pl.ds(s * 16, 16)]
            wiv = wib[pl.ds(s * 16, 16)]
            svalid = s < offw
            for bit in range(16):
                isset = ((lax.shift_right_logical(wvec, bit) & 1) != 0) & svalid
                col = (wiv * 16 + bit) & (n - 1)
                ok = isset & (off < CAP)
                tgt = off * 16 + lane
                plsc.store_scatter(li, [tgt], col, mask=ok)
                off = off + ok.astype(jnp.int32)
            return off

        off = lax.fori_loop(0, nmax, slot_body, jnp.zeros((16,), jnp.int32))

        rfb = (base + r) * n
        for s2 in range(K // 16):
            tgtp = r * K + s2 * 16 + lane
            plsc.store_scatter(oi, [tgtp], jnp.full((16,), base + r, jnp.int32))
            plsc.store_scatter(
                fi2d, [lax.shift_right_logical(tgtp, 7), tgtp & 127],
                jnp.full((16,), rfb + base + r, jnp.int32))

        excl = plsc.cumsum(off) - off
        for s in range(K):
            vi = li[pl.ds(s * 16, 16)]
            m2 = (s < off) & (excl + s < K)
            tgt = r * K + excl + s
            plsc.store_scatter(oi, [tgt], vi, mask=m2)
            plsc.store_scatter(
                fi2d, [lax.shift_right_logical(tgt, 7), tgt & 127],
                rfb + vi, mask=m2)

    def pair_body(r2, _):
        do_row(r2 * 2, 0)
        do_row(r2 * 2 + 1, 1)
        return 0

    lax.fori_loop(0, rpw // 2, pair_body, 0)

    nq = rpw * K // 128

    def fire(q):
        pltpu.async_copy(mf_hbm.at[fi2d.at[q]], ow.at[pl.ds(q * 128, 128)],
                         gsem)

    def drain(q):
        pltpu.make_async_copy(mf_hbm.at[fi2d.at[q]],
                              ow.at[pl.ds(q * 128, 128)], gsem).wait()

    def group_body(g, _):
        for qq in range(8):
            fire(g * 8 + qq)
        for qq in range(8):
            drain(g * 8 + qq)
        return 0

    lax.fori_loop(0, nq // 8, group_body, 0)

    pltpu.sync_copy(oi, idx_hbm.at[pl.ds(base * K, rpw * K)])
    pltpu.sync_copy(ow, w_hbm.at[pl.ds(base * K, rpw * K)])


def _gather_body(hid_hbm, idx_hbm, w_hbm, agg_hbm, iv, wv, gb0, gb1,
                 ob0, ob1, gs0, gs1, os0, os1, *, h, rpw):
    wid = lax.axis_index("s") * NC + lax.axis_index("c")
    base = wid * rpw
    gbufs = (gb0, gb1)
    gsems = (gs0, gs1)
    obufs = (ob0, ob1)
    osems = (os0, os1)
    obl = 16

    pltpu.sync_copy(idx_hbm.at[pl.ds(base * K, rpw * K)], iv)
    pltpu.sync_copy(w_hbm.at[pl.ds(base * K, rpw * K)], wv)

    def start_gather(r, u):
        pltpu.async_copy(hid_hbm.at[iv.at[pl.ds(r * K, K)]], gbufs[u], gsems[u])

    for u in range(2):
        start_gather(u, u)

    def do_row(r, rr, u, ob):
        pltpu.make_async_copy(
            hid_hbm.at[iv.at[pl.ds(r * K, K)]], gbufs[u], gsems[u]
        ).wait()
        rows = gbufs[u]

        def nb_body(qq, acc):
            acc = list(acc)
            for uu in range(8):
                nnb = qq * 8 + uu
                wn = plsc.load_gather(
                    wv, [jnp.full((16,), r * K + nnb, jnp.int32)]
                )
                for c in range(h // 16):
                    acc[c] = acc[c] + wn * rows[nnb, pl.ds(c * 16, 16)]
            return tuple(acc)

        acc = lax.fori_loop(
            0, K // 8, nb_body,
            tuple(jnp.zeros((16,), jnp.float32) for _ in range(h // 16)),
        )
        for c in range(h // 16):
            ob[pl.ds(rr * h + c * 16, 16)] = acc[c]

        @pl.when(r + 2 < rpw)
        def _refill():
            start_gather(r + 2, u)

    def block_pair(b2, _):
        for j in range(2):
            blk = b2 * 2 + j

            @pl.when(blk >= 2)
            def _drain_prev():
                pltpu.make_async_copy(
                    obufs[j],
                    agg_hbm.at[pl.ds((base + (blk - 2) * obl) * h, obl * h)],
                    osems[j],
                ).wait()

            def q_body(q, _):
                for u3 in range(2):
                    rr = q * 2 + u3
                    r = blk * obl + rr
                    do_row(r, rr, u3, obufs[j])
                return 0

            lax.fori_loop(0, obl // 2, q_body, 0)
            pltpu.async_copy(
                obufs[j],
                agg_hbm.at[pl.ds((base + blk * obl) * h, obl * h)],
                osems[j],
            )
        return 0

    lax.fori_loop(0, rpw // (2 * obl), block_pair, 0)

    for j in range(2):
        blk = rpw // obl - 2 + j
        pltpu.make_async_copy(
            obufs[j],
            agg_hbm.at[pl.ds((base + blk * obl) * h, obl * h)],
            osems[j],
        ).wait()


def kernel(x, W_in, b_in, W_m1, b_m1, W_m2, b_m2, ln_g, ln_b, W_r1, b_r1,
           W_r2, b_r2):
    n, f = x.shape
    h = W_in.shape[1]
    rb = min(128, n)
    grid = (n // rb,)
    rpw = n // NW

    full = lambda shape: pl.BlockSpec(shape, lambda i: (0,) * len(shape))
    rowblk = lambda shape: pl.BlockSpec(shape, lambda i: (i,) + (0,) * (len(shape) - 1))

    W1h, W1a, W1x = W_m1[:h], W_m1[h:2 * h], W_m1[2 * h:]

    jj = jnp.arange(n)
    P = jnp.zeros((n, n // 16), jnp.float32).at[jj, jj // 16].set(
        (2.0 ** (jj % 16)).astype(jnp.float32))

    M, PB, hidden, xterm = pl.pallas_call(
        functools.partial(_knn_body, rb=rb, n=n, k=K),
        grid=grid,
        in_specs=[full((n, f)), full((f, h)), full((h,)), full((f, h)),
                  full((h,)), full((n, n // 16))],
        out_specs=[rowblk((rb, n)), rowblk((rb, n // 16)), rowblk((rb, h)),
                   rowblk((rb, h))],
        out_shape=[
            jax.ShapeDtypeStruct((n, n), jnp.float32),
            jax.ShapeDtypeStruct((n, n // 16), jnp.float32),
            jax.ShapeDtypeStruct((n, h), jnp.float32),
            jax.ShapeDtypeStruct((n, h), jnp.float32),
        ],
    )(x, W_in, b_in, W1x, b_m1, P)

    mesh = plsc.VectorSubcoreMesh(
        core_axis_name="c", subcore_axis_name="s", num_cores=NC,
        num_subcores=NS,
    )

    sc_params = pltpu.CompilerParams(needs_layout_passes=False)

    idxs, ws = pl.kernel(
        functools.partial(_compact_body, n=n, rpw=rpw),
        compiler_params=sc_params,
        out_type=[
            jax.ShapeDtypeStruct((n * K,), jnp.int32),
            jax.ShapeDtypeStruct((n * K,), jnp.float32),
        ],
        mesh=mesh,
        scratch_types=[
            pltpu.VMEM((n // 16,), jnp.float32),
            pltpu.VMEM((n // 16,), jnp.float32),
            pltpu.VMEM((CAP * 16,), jnp.int32),
            pltpu.VMEM((CAP * 16,), jnp.int32),
            pltpu.VMEM((CAP * 16,), jnp.int32),
            pltpu.VMEM((rpw * K // 128, 128), jnp.int32),
            pltpu.VMEM((rpw * K,), jnp.int32),
            pltpu.VMEM((rpw * K,), jnp.float32),
            pltpu.SemaphoreType.DMA,
            pltpu.SemaphoreType.DMA,
            pltpu.SemaphoreType.DMA,
        ],
    )(M.reshape(n * n), PB)

    gather = pl.kernel(
        functools.partial(_gather_body, h=h, rpw=rpw),
        compiler_params=sc_params,
        out_type=jax.ShapeDtypeStruct((n * h,), jnp.float32),
        mesh=mesh,
        scratch_types=(
            [pltpu.VMEM((rpw * K,), jnp.int32),
             pltpu.VMEM((rpw * K,), jnp.float32)]
            + [pltpu.VMEM((K, h), jnp.float32) for _ in range(2)]
            + [pltpu.VMEM((16 * h,), jnp.float32) for _ in range(2)]
            + [pltpu.SemaphoreType.DMA for _ in range(4)]
        ),
    )

    step = pl.pallas_call(
        _step_body,
        grid=grid,
        in_specs=[rowblk((rb, h)), rowblk((rb, h)), rowblk((rb, h)),
                  full((h, h)), full((h, h)), full((h, h)), full((h,)),
                  full((h,)), full((h,))],
        out_specs=rowblk((rb, h)),
        out_shape=jax.ShapeDtypeStruct((n, h), jnp.float32),
    )
    for _ in range(STEPS):
        agg = gather(hidden, idxs, ws).reshape(n, h)
        hidden = step(agg, hidden, xterm, W1h, W1a, W_m2, b_m2, ln_g, ln_b)

    Wr1h, Wr1x = W_r1[:h], W_r1[h:]
    out = pl.pallas_call(
        _readout_body,
        grid=grid,
        in_specs=[rowblk((rb, h)), rowblk((rb, f)), full((h, h)), full((f, h)),
                  full((h,)), full((h, 1)), full((1,))],
        out_specs=rowblk((rb,)),
        out_shape=jax.ShapeDtypeStruct((n,), jnp.float32),
    )(hidden, x, Wr1h, Wr1x, b_r1, W_r2, b_r2)
    return out


# speedup vs baseline: 1.0421x; 1.0421x over previous
import functools

import jax
import jax.numpy as jnp
from jax import lax
from jax.experimental import pallas as pl
from jax.experimental.pallas import tpu as pltpu
from jax.experimental.pallas import tpu_sc as plsc

K = 32
TEMP = 0.1
STEPS = 3
LN_EPS = 1e-5

NC = 2
NS = 16
NW = NC * NS
CAP = 34


def _silu(v):
    return v * (1.0 / (1.0 + jnp.exp(-v)))


def _knn_body(x_ref, w_in_ref, b_in_ref, w1x_ref, b1_ref, p_ref, m_ref,
              pb_ref, h0_ref, xt_ref, *, rb, n, k):
    xb = x_ref[pl.ds(pl.program_id(0) * rb, rb), :]
    xall = x_ref[...]
    nb = jnp.sum(xb * xb, axis=1, keepdims=True)
    na = jnp.sum(xall * xall, axis=1)[None, :]
    cross = lax.dot_general(
        xb, xall, (((1,), (1,)), ((), ())), preferred_element_type=jnp.float32
    )
    sq = jnp.maximum(nb + na - 2.0 * cross, 0.0)
    dist = jnp.sqrt(jnp.maximum(sq, 1e-12))
    rows = pl.program_id(0) * rb + lax.broadcasted_iota(jnp.int32, (rb, n), 0)
    cols = lax.broadcasted_iota(jnp.int32, (rb, n), 1)
    dist = jnp.where(rows == cols, jnp.inf, dist)

    bits = lax.bitcast_convert_type(dist, jnp.int32)

    def search_body(_, lohi):
        lo, hi = lohi
        mid = lo + lax.div(hi - lo, 2)
        cnt = jnp.sum((bits <= mid[:, None]).astype(jnp.float32), axis=1)
        ge = cnt >= float(k)
        return jnp.where(ge, lo, mid + 1), jnp.where(ge, mid, hi)

    lo0 = jnp.zeros((rb,), jnp.int32)
    hi0 = jnp.full((rb,), jnp.int32(0x7F800000))
    _, hi = lax.fori_loop(0, 31, search_body, (lo0, hi0))
    t = lax.bitcast_convert_type(hi, jnp.float32)

    m = jnp.min(dist, axis=1, keepdims=True)
    mask = dist <= t[:, None]
    e = jnp.where(mask, jnp.exp(-(dist - m) / TEMP), 0.0)
    den = jnp.sum(e, axis=1, keepdims=True)
    m_ref[...] = e / den

    def mm(a, b):
        return lax.dot_general(
            a, b, (((1,), (0,)), ((), ())), preferred_element_type=jnp.float32
        )

    pb_ref[...] = mm(mask.astype(jnp.float32), p_ref[...])

    h0_ref[...] = mm(xb, w_in_ref[...]) + b_in_ref[...]
    xt_ref[...] = mm(xb, w1x_ref[...]) + b1_ref[...]


def _step_body(agg_ref, hb_ref, xt_ref, w1h_ref, w1a_ref, w2_ref, b2_ref,
               g_ref, b_ref, out_ref):
    def mm(a, b):
        return lax.dot_general(
            a, b, (((1,), (0,)), ((), ())), preferred_element_type=jnp.float32
        )

    hb = hb_ref[...]
    z = mm(hb, w1h_ref[...]) + mm(agg_ref[...], w1a_ref[...]) + xt_ref[...]
    msg = mm(_silu(z), w2_ref[...]) + b2_ref[...]
    pre = hb + msg
    mu = jnp.mean(pre, axis=1, keepdims=True)
    var = jnp.mean((pre - mu) ** 2, axis=1, keepdims=True)
    out_ref[...] = (pre - mu) / jnp.sqrt(var + LN_EPS) * g_ref[...] + b_ref[...]


def _readout_body(hb_ref, xb_ref, w1h_ref, w1x_ref, b1_ref, w2_ref, b2_ref,
                  out_ref):
    def mm(a, b):
        return lax.dot_general(
            a, b, (((1,), (0,)), ((), ())), preferred_element_type=jnp.float32
        )

    z = mm(hb_ref[...], w1h_ref[...]) + mm(xb_ref[...], w1x_ref[...]) + b1_ref[...]
    h1 = _silu(z)
    ro = jnp.sum(h1 * w2_ref[...].T, axis=1) + b2_ref[0]
    out_ref[...] = jnp.maximum(ro, 0.0) + jnp.log1p(jnp.exp(-jnp.abs(ro)))


def _compact_body(m_hbm, pb_hbm, idx_hbm, w_hbm, mrow0, mrow1, pb0, pb1,
                  wvb, wib, li, lw, oi, ow, sem0, sem1, psem0, psem1,
                  *, n, rpw):
    wid = lax.axis_index("s") * NC + lax.axis_index("c")
    base = wid * rpw
    lane = lax.iota(jnp.int32, 16)
    sems = (sem0, sem1)
    psems = (psem0, psem1)
    bufs = (mrow0, mrow1)
    pbufs = (pb0, pb1)
    nw16 = n // 16

    def zero_body(q, _):
        ow[pl.ds(q * 16, 16)] = jnp.zeros((16,), jnp.float32)
        oi[pl.ds(q * 16, 16)] = jnp.zeros((16,), jnp.int32)
        return 0

    lax.fori_loop(0, rpw * K // 16, zero_body, 0)

    pltpu.async_copy(m_hbm.at[base], mrow0, sem0)
    pltpu.async_copy(pb_hbm.at[base], pb0, psem0)

    def do_row(r, b):
        @pl.when(r + 1 < rpw)
        def _start_next():
            pltpu.async_copy(m_hbm.at[base + r + 1], bufs[1 - b], sems[1 - b])
            pltpu.async_copy(pb_hbm.at[base + r + 1], pbufs[1 - b],
                             psems[1 - b])

        pltpu.make_async_copy(m_hbm.at[base + r], bufs[b], sems[b]).wait()
        pltpu.make_async_copy(pb_hbm.at[base + r], pbufs[b], psems[b]).wait()
        buf = bufs[b]
        pbuf = pbufs[b]

        def word_body(c4, offw):
            for u in range(4):
                c = c4 * 4 + u
                wq = pbuf[pl.ds(c * 16, 16)].astype(jnp.int32)
                nz = (wq != 0) & (offw < CAP)
                tgt = offw * 16 + lane
                plsc.store_scatter(wib, [tgt], c * 16 + lane, mask=nz)
                plsc.store_scatter(wvb, [tgt], wq, mask=nz)
                offw = offw + nz.astype(jnp.int32)
            return offw

        offw = lax.fori_loop(0, nw16 // 64, word_body,
                             jnp.zeros((16,), jnp.int32))
        nmax = lax.reduce_max(offw, (0,))

        def slot_body(s, off):
            wvec = wvb[pl.ds(s * 16, 16)]
            wiv = wib[pl.ds(s * 16, 16)]
            svalid = s < offw
            for bit in range(16):
                isset = ((lax.shift_right_logical(wvec, bit) & 1) != 0) & svalid
                col = (wiv * 16 + bit) & (n - 1)
                mval = plsc.load_gather(buf, [col], mask=isset)
                ok = isset & (off < CAP)
                tgt = off * 16 + lane
                plsc.store_scatter(li, [tgt], col, mask=ok)
                plsc.store_scatter(lw, [tgt], mval, mask=ok)
                off = off + ok.astype(jnp.int32)
            return off

        off = lax.fori_loop(0, nmax, slot_body, jnp.zeros((16,), jnp.int32))

        excl = plsc.cumsum(off) - off
        for s in range(K):
            vi = li[pl.ds(s * 16, 16)]
            vw = lw[pl.ds(s * 16, 16)]
            m2 = (s < off) & (excl + s < K)
            tgt = r * K + excl + s
            plsc.store_scatter(oi, [tgt], vi, mask=m2)
            plsc.store_scatter(ow, [tgt], vw, mask=m2)

    def pair_body(r2, _):
        do_row(r2 * 2, 0)
        do_row(r2 * 2 + 1, 1)
        return 0

    lax.fori_loop(0, rpw // 2, pair_body, 0)

    pltpu.sync_copy(oi, idx_hbm.at[pl.ds(base * K, rpw * K)])
    pltpu.sync_copy(ow, w_hbm.at[pl.ds(base * K, rpw * K)])


def _gather_body(hid_hbm, idx_hbm, w_hbm, agg_hbm, iv, wv, gb0, gb1,
                 ob0, ob1, gs0, gs1, os0, os1, *, h, rpw):
    wid = lax.axis_index("s") * NC + lax.axis_index("c")
    base = wid * rpw
    gbufs = (gb0, gb1)
    gsems = (gs0, gs1)
    obufs = (ob0, ob1)
    osems = (os0, os1)
    obl = 16

    pltpu.sync_copy(idx_hbm.at[pl.ds(base * K, rpw * K)], iv)
    pltpu.sync_copy(w_hbm.at[pl.ds(base * K, rpw * K)], wv)

    def start_gather(r, u):
        pltpu.async_copy(hid_hbm.at[iv.at[pl.ds(r * K, K)]], gbufs[u], gsems[u])

    for u in range(2):
        start_gather(u, u)

    def do_row(r, rr, u, ob):
        pltpu.make_async_copy(
            hid_hbm.at[iv.at[pl.ds(r * K, K)]], gbufs[u], gsems[u]
        ).wait()
        rows = gbufs[u]

        def nb_body(qq, acc):
            acc = list(acc)
            for uu in range(8):
                nnb = qq * 8 + uu
                wn = plsc.load_gather(
                    wv, [jnp.full((16,), r * K + nnb, jnp.int32)]
                )
                for c in range(h // 16):
                    acc[c] = acc[c] + wn * rows[nnb, pl.ds(c * 16, 16)]
            return tuple(acc)

        acc = lax.fori_loop(
            0, K // 8, nb_body,
            tuple(jnp.zeros((16,), jnp.float32) for _ in range(h // 16)),
        )
        for c in range(h // 16):
            ob[pl.ds(rr * h + c * 16, 16)] = acc[c]

        @pl.when(r + 2 < rpw)
        def _refill():
            start_gather(r + 2, u)

    def block_pair(b2, _):
        for j in range(2):
            blk = b2 * 2 + j

            @pl.when(blk >= 2)
            def _drain_prev():
                pltpu.make_async_copy(
                    obufs[j],
                    agg_hbm.at[pl.ds((base + (blk - 2) * obl) * h, obl * h)],
                    osems[j],
                ).wait()

            def q_body(q, _):
                for u3 in range(2):
                    rr = q * 2 + u3
                    r = blk * obl + rr
                    do_row(r, rr, u3, obufs[j])
                return 0

            lax.fori_loop(0, obl // 2, q_body, 0)
            pltpu.async_copy(
                obufs[j],
                agg_hbm.at[pl.ds((base + blk * obl) * h, obl * h)],
                osems[j],
            )
        return 0

    lax.fori_loop(0, rpw // (2 * obl), block_pair, 0)

    for j in range(2):
        blk = rpw // obl - 2 + j
        pltpu.make_async_copy(
            obufs[j],
            agg_hbm.at[pl.ds((base + blk * obl) * h, obl * h)],
            osems[j],
        ).wait()


def kernel(x, W_in, b_in, W_m1, b_m1, W_m2, b_m2, ln_g, ln_b, W_r1, b_r1,
           W_r2, b_r2):
    n, f = x.shape
    h = W_in.shape[1]
    rb = min(128, n)
    grid = (n // rb,)
    rpw = n // NW

    full = lambda shape: pl.BlockSpec(shape, lambda i: (0,) * len(shape))
    rowblk = lambda shape: pl.BlockSpec(shape, lambda i: (i,) + (0,) * (len(shape) - 1))

    W1h, W1a, W1x = W_m1[:h], W_m1[h:2 * h], W_m1[2 * h:]

    jj = jnp.arange(n)
    P = jnp.zeros((n, n // 16), jnp.float32).at[jj, jj // 16].set(
        (2.0 ** (jj % 16)).astype(jnp.float32))

    M, PB, hidden, xterm = pl.pallas_call(
        functools.partial(_knn_body, rb=rb, n=n, k=K),
        grid=grid,
        in_specs=[full((n, f)), full((f, h)), full((h,)), full((f, h)),
                  full((h,)), full((n, n // 16))],
        out_specs=[rowblk((rb, n)), rowblk((rb, n // 16)), rowblk((rb, h)),
                   rowblk((rb, h))],
        out_shape=[
            jax.ShapeDtypeStruct((n, n), jnp.float32),
            jax.ShapeDtypeStruct((n, n // 16), jnp.float32),
            jax.ShapeDtypeStruct((n, h), jnp.float32),
            jax.ShapeDtypeStruct((n, h), jnp.float32),
        ],
    )(x, W_in, b_in, W1x, b_m1, P)

    mesh = plsc.VectorSubcoreMesh(
        core_axis_name="c", subcore_axis_name="s", num_cores=NC,
        num_subcores=NS,
    )

    sc_params = pltpu.CompilerParams(needs_layout_passes=False)

    idxs, ws = pl.kernel(
        functools.partial(_compact_body, n=n, rpw=rpw),
        compiler_params=sc_params,
        out_type=[
            jax.ShapeDtypeStruct((n * K,), jnp.int32),
            jax.ShapeDtypeStruct((n * K,), jnp.float32),
        ],
        mesh=mesh,
        scratch_types=[
            pltpu.VMEM((n,), jnp.float32),
            pltpu.VMEM((n,), jnp.float32),
            pltpu.VMEM((n // 16,), jnp.float32),
            pltpu.VMEM((n // 16,), jnp.float32),
            pltpu.VMEM((CAP * 16,), jnp.int32),
            pltpu.VMEM((CAP * 16,), jnp.int32),
            pltpu.VMEM((CAP * 16,), jnp.int32),
            pltpu.VMEM((CAP * 16,), jnp.float32),
            pltpu.VMEM((rpw * K,), jnp.int32),
            pltpu.VMEM((rpw * K,), jnp.float32),
            pltpu.SemaphoreType.DMA,
            pltpu.SemaphoreType.DMA,
            pltpu.SemaphoreType.DMA,
            pltpu.SemaphoreType.DMA,
        ],
    )(M, PB)

    gather = pl.kernel(
        functools.partial(_gather_body, h=h, rpw=rpw),
        compiler_params=sc_params,
        out_type=jax.ShapeDtypeStruct((n * h,), jnp.float32),
        mesh=mesh,
        scratch_types=(
            [pltpu.VMEM((rpw * K,), jnp.int32),
             pltpu.VMEM((rpw * K,), jnp.float32)]
            + [pltpu.VMEM((K, h), jnp.float32) for _ in range(2)]
            + [pltpu.VMEM((16 * h,), jnp.float32) for _ in range(2)]
            + [pltpu.SemaphoreType.DMA for _ in range(4)]
        ),
    )

    step = pl.pallas_call(
        _step_body,
        grid=grid,
        in_specs=[rowblk((rb, h)), rowblk((rb, h)), rowblk((rb, h)),
                  full((h, h)), full((h, h)), full((h, h)), full((h,)),
                  full((h,)), full((h,))],
        out_specs=rowblk((rb, h)),
        out_shape=jax.ShapeDtypeStruct((n, h), jnp.float32),
    )
    for _ in range(STEPS):
        agg = gather(hidden, idxs, ws).reshape(n, h)
        hidden = step(agg, hidden, xterm, W1h, W1a, W_m2, b_m2, ln_g, ln_b)

    Wr1h, Wr1x = W_r1[:h], W_r1[h:]
    out = pl.pallas_call(
        _readout_body,
        grid=grid,
        in_specs=[rowblk((rb, h)), rowblk((rb, f)), full((h, h)), full((f, h)),
                  full((h,)), full((h, 1)), full((1,))],
        out_specs=rowblk((rb,)),
        out_shape=jax.ShapeDtypeStruct((n,), jnp.float32),
    )(hidden, x, Wr1h, Wr1x, b_r1, W_r2, b_r2)
    return out
